# baseline (speedup 1.0000x reference)
import functools

import jax
import jax.numpy as jnp
from jax import lax
from jax.experimental import pallas as pl
from jax.experimental.pallas import tpu as pltpu
from jax.experimental.pallas import tpu_sc as plsc

B = 16384
D = 32
N_NUM = 13
N_CAT = 26
PER_FIELD = 100000

NC = 2
NS = 16
NW = NC * NS
ROWS_PER_W = B // NW
CB = 32
NCHUNK = ROWS_PER_W // CB


def _body(x_hbm, nw_hbm, emb_hbm, bias_hbm, out_hbm,
          x_v, idx_v, stage_v, out_c, nw_v, bias_v, sem):
    wid = lax.axis_index("s") * NC + lax.axis_index("c")
    pltpu.sync_copy(nw_hbm, nw_v)
    pltpu.sync_copy(bias_hbm, bias_v)

    iota = lax.iota(jnp.int32, 16)
    off0 = iota * PER_FIELD
    off1 = (iota + 10) * PER_FIELD

    def chunk(c, _):
        base = wid * ROWS_PER_W + c * CB
        pltpu.sync_copy(x_hbm.at[pl.ds(base, CB)], x_v)

        def idx_row(b, carry):
            v0 = x_v[b, pl.ds(13, 16)].astype(jnp.int32) + off0
            v1 = x_v[b, pl.ds(23, 16)].astype(jnp.int32) + off1
            idx_v[b, pl.ds(0, 16)] = v0
            idx_v[b, pl.ds(10, 16)] = v1
            return carry
        lax.fori_loop(0, CB, idx_row, None)

        copies = [
            pltpu.async_copy(emb_hbm.at[idx_v.at[b]],
                             stage_v.at[pl.ds(b * N_CAT, N_CAT)], sem)
            for b in range(CB)
        ]

        def num_row(b, carry):
            xrow = x_v[b, pl.ds(0, 16)]
            out_c[b, 0, pl.ds(0, 16)] = nw_v[0, pl.ds(0, 16)]
            out_c[b, 0, pl.ds(16, 16)] = nw_v[0, pl.ds(16, 16)]
            for j in range(1, N_NUM + 1):
                s = xrow[j - 1]
                for h in range(2):
                    sl = pl.ds(h * 16, 16)
                    out_c[b, j, sl] = nw_v[j, sl] * s + bias_v[j - 1, sl]
            return carry
        lax.fori_loop(0, CB, num_row, None)

        for cp in copies:
            cp.wait()

        bks = [(bias_v[N_NUM + k, pl.ds(0, 16)], bias_v[N_NUM + k, pl.ds(16, 16)])
               for k in range(N_CAT)]

        def cat_row(b, carry):
            r0 = b * N_CAT
            for k in range(N_CAT):
                bk0, bk1 = bks[k]
                out_c[b, N_NUM + 1 + k, pl.ds(0, 16)] = stage_v[r0 + k, pl.ds(0, 16)] + bk0
                out_c[b, N_NUM + 1 + k, pl.ds(16, 16)] = stage_v[r0 + k, pl.ds(16, 16)] + bk1
            return carry
        lax.fori_loop(0, CB, cat_row, None)

        pltpu.sync_copy(out_c, out_hbm.at[pl.ds(base, CB)])
        return _

    lax.fori_loop(0, NCHUNK, chunk, None)


@jax.jit
def kernel(x, num_weight, emb_table, bias):
    run = pl.kernel(
        _body,
        out_type=jax.ShapeDtypeStruct((B, N_NUM + 1 + N_CAT, D), jnp.float32),
        mesh=plsc.VectorSubcoreMesh(core_axis_name="c", subcore_axis_name="s"),
        compiler_params=pltpu.CompilerParams(use_tc_tiling_on_sc=False),
        scratch_types=[
            pltpu.VMEM((CB, N_NUM + N_CAT), jnp.float32),
            pltpu.VMEM((CB, N_CAT), jnp.int32),
            pltpu.VMEM((N_CAT * CB, D), jnp.float32),
            pltpu.VMEM((CB, N_NUM + 1 + N_CAT, D), jnp.float32),
            pltpu.VMEM((N_NUM + 1, D), jnp.float32),
            pltpu.VMEM((N_NUM + N_CAT, D), jnp.float32),
            pltpu.SemaphoreType.DMA,
        ],
    )
    return run(x, num_weight, emb_table, bias)
